# baseline (speedup 1.0000x reference)
import functools

import jax
import jax.numpy as jnp
from jax import lax
from jax.experimental import pallas as pl
from jax.experimental.pallas import tpu as pltpu
from jax.experimental.pallas import tpu_sc as plsc

_NUM_CORES = 2
_NUM_SUBCORES = 16
_NUM_WORKERS = _NUM_CORES * _NUM_SUBCORES


def _make_gather(b: int, p: int, d: int, chunk: int, nbuf: int):
    num_rows = b * p
    assert num_rows % (_NUM_WORKERS * chunk) == 0
    rows_per_worker = num_rows // _NUM_WORKERS
    assert p % rows_per_worker == 0
    n_chunks = rows_per_worker // chunk
    assert nbuf <= n_chunks
    mesh = plsc.VectorSubcoreMesh(
        core_axis_name="c", subcore_axis_name="s", num_cores=_NUM_CORES
    )

    @functools.partial(
        pl.kernel,
        mesh=mesh,
        out_type=jax.ShapeDtypeStruct((num_rows, d), jnp.float32),
        scratch_types=[
            pltpu.VMEM((rows_per_worker,), jnp.int32),
            pltpu.VMEM((nbuf, chunk, d), jnp.float32),
            pltpu.SemaphoreType.DMA,
            pltpu.SemaphoreType.DMA,
        ],
    )
    def gather_kernel(table_hbm, x_hbm, out_hbm, idx_v, rows_v, gsem, wsem):
        wid = lax.axis_index("s") * _NUM_CORES + lax.axis_index("c")
        base = wid * rows_per_worker
        row = wid // (p // rows_per_worker)
        col = (wid % (p // rows_per_worker)) * rows_per_worker
        pltpu.sync_copy(x_hbm.at[row, pl.ds(col, rows_per_worker)], idx_v)

        def start_g(c):
            return pltpu.async_copy(
                table_hbm.at[idx_v.at[pl.ds(c * chunk, chunk)]],
                rows_v.at[c % nbuf],
                gsem,
            )

        def start_w(c):
            return pltpu.async_copy(
                rows_v.at[c % nbuf],
                out_hbm.at[pl.ds(base + c * chunk, chunk)],
                wsem,
            )

        g = [None] * n_chunks
        w = [None] * n_chunks
        for j in range(nbuf):
            g[j] = start_g(j)
        for c in range(n_chunks):
            g[c].wait()
            w[c] = start_w(c)
            j = c - 1 + nbuf
            if c >= 1 and j < n_chunks:
                w[c - 1].wait()
                g[j] = start_g(j)
        for c in range(n_chunks):
            if not (1 <= c + 1 and c + nbuf < n_chunks):
                w[c].wait()

    return gather_kernel


@jax.jit
def kernel(x, W_E):
    d, v = W_E.shape
    b, p = x.shape
    W_T = W_E.T
    gather = _make_gather(b, p, d, chunk=32, nbuf=4)
    out = gather(W_T, x.astype(jnp.int32))
    return out.reshape(b, p, d)


# speedup vs baseline: 1.0051x; 1.0051x over previous
import functools

import jax
import jax.numpy as jnp
from jax import lax
from jax.experimental import pallas as pl
from jax.experimental.pallas import tpu as pltpu
from jax.experimental.pallas import tpu_sc as plsc

_NUM_CORES = 2
_NUM_SUBCORES = 16
_NUM_WORKERS = _NUM_CORES * _NUM_SUBCORES


def _make_gather(b: int, p: int, d: int, chunk: int, nbuf: int):
    num_rows = b * p
    assert num_rows % (_NUM_WORKERS * chunk) == 0
    rows_per_worker = num_rows // _NUM_WORKERS
    assert p % rows_per_worker == 0
    n_chunks = rows_per_worker // chunk
    assert nbuf <= n_chunks
    mesh = plsc.VectorSubcoreMesh(
        core_axis_name="c", subcore_axis_name="s", num_cores=_NUM_CORES
    )

    @functools.partial(
        pl.kernel,
        mesh=mesh,
        out_type=jax.ShapeDtypeStruct((num_rows, d), jnp.float32),
        scratch_types=[
            pltpu.VMEM((rows_per_worker,), jnp.int32),
            pltpu.VMEM((nbuf, chunk, d), jnp.float32),
            pltpu.SemaphoreType.DMA,
            pltpu.SemaphoreType.DMA,
        ],
    )
    def gather_kernel(table_hbm, x_hbm, out_hbm, idx_v, rows_v, gsem, wsem):
        wid = lax.axis_index("s") * _NUM_CORES + lax.axis_index("c")
        base = wid * rows_per_worker
        row = wid // (p // rows_per_worker)
        col = (wid % (p // rows_per_worker)) * rows_per_worker
        pltpu.sync_copy(x_hbm.at[row, pl.ds(col, rows_per_worker)], idx_v)

        def start_g(c):
            return pltpu.async_copy(
                table_hbm.at[idx_v.at[pl.ds(c * chunk, chunk)]],
                rows_v.at[c % nbuf],
                gsem,
            )

        def start_w(c):
            return pltpu.async_copy(
                rows_v.at[c % nbuf],
                out_hbm.at[pl.ds(base + c * chunk, chunk)],
                wsem,
            )

        g = [None] * n_chunks
        w = [None] * n_chunks
        for j in range(nbuf):
            g[j] = start_g(j)
        for c in range(n_chunks):
            g[c].wait()
            w[c] = start_w(c)
            j = c - 1 + nbuf
            if c >= 1 and j < n_chunks:
                w[c - 1].wait()
                g[j] = start_g(j)
        for c in range(n_chunks):
            if c + nbuf >= n_chunks:
                w[c].wait()

    return gather_kernel


@jax.jit
def kernel(x, W_E):
    d, v = W_E.shape
    b, p = x.shape
    W_T = W_E.T
    gather = _make_gather(b, p, d, chunk=32, nbuf=4)
    out = gather(W_T, x.astype(jnp.int32))
    return out.reshape(b, p, d)
